# baseline (speedup 1.0000x reference)
import functools

import jax
import jax.numpy as jnp
from jax import lax
from jax.experimental import pallas as pl
from jax.experimental.pallas import tpu as pltpu
from jax.experimental.pallas import tpu_sc as plsc

B, F, V, E, H = 16384, 26, 100000, 128, 512
NC, NS, L = 2, 16, 16
NW = NC * NS
NSLICE = 4
BS = B // NSLICE
CHUNK = 16
RPC = CHUNK * F
GSLICE = 104
NG = RPC // GSLICE

_sc_mesh = plsc.VectorSubcoreMesh(core_axis_name="c", subcore_axis_name="s")


def _make_sc_gather_reduce(bs):
    s_per_w = bs // NW
    n_chunk = s_per_w // CHUNK

    @functools.partial(
        pl.kernel,
        out_type=(
            jax.ShapeDtypeStruct((bs, E), jnp.float32),
            jax.ShapeDtypeStruct((bs, E), jnp.float32),
            jax.ShapeDtypeStruct((bs * F,), jnp.float32),
        ),
        mesh=_sc_mesh,
        compiler_params=pltpu.CompilerParams(needs_layout_passes=False),
        scratch_types=[
            pltpu.VMEM((RPC,), jnp.int32),
            pltpu.VMEM((RPC,), jnp.int32),
            pltpu.VMEM((RPC,), jnp.float32),
            pltpu.VMEM((RPC,), jnp.float32),
            pltpu.VMEM((RPC,), jnp.float32),
            pltpu.VMEM((RPC,), jnp.float32),
            pltpu.VMEM((RPC, E), jnp.float32),
            pltpu.VMEM((RPC, E), jnp.float32),
            pltpu.VMEM((CHUNK, E), jnp.float32),
            pltpu.VMEM((CHUNK, E), jnp.float32),
            pltpu.VMEM((CHUNK, E), jnp.float32),
            pltpu.VMEM((CHUNK, E), jnp.float32),
            pltpu.SemaphoreType.DMA,
            pltpu.SemaphoreType.DMA,
            pltpu.SemaphoreType.DMA,
            pltpu.SemaphoreType.DMA,
            pltpu.SemaphoreType.DMA,
            pltpu.SemaphoreType.DMA,
            pltpu.SemaphoreType.DMA,
            pltpu.SemaphoreType.DMA,
        ],
    )
    def sc_gather_reduce(gidx_hbm, xv_hbm, emb1_hbm, emb2_hbm,
                         sum_hbm, sq_hbm, fo_hbm,
                         idx_v0, idx_v1, xv_v0, xv_v1, fo_v0, fo_v1,
                         rows_v0, rows_v1, sum_st0, sum_st1, sq_st0, sq_st1,
                         i_sem0, i_sem1, r_sem0, r_sem1, f_sem0, f_sem1,
                         o_sem0, o_sem1):
        idx_v = (idx_v0, idx_v1)
        xv_v = (xv_v0, xv_v1)
        fo_v = (fo_v0, fo_v1)
        rows_v = (rows_v0, rows_v1)
        sum_st = (sum_st0, sum_st1)
        sq_st = (sq_st0, sq_st1)
        i_sems = (i_sem0, i_sem1)
        r_sems = (r_sem0, r_sem1)
        f_sems = (f_sem0, f_sem1)
        o_sems = (o_sem0, o_sem1)
        wid = lax.axis_index("s") * NC + lax.axis_index("c")
        base = wid * s_per_w

        def fire_idx(b, c):
            cb = (base + c * CHUNK) * F
            pltpu.async_copy(gidx_hbm.at[pl.ds(cb, RPC)], idx_v[b], i_sems[b])
            pltpu.async_copy(xv_hbm.at[pl.ds(cb, RPC)], xv_v[b], i_sems[b])

        def wait_idx(b):
            pltpu.make_async_copy(gidx_hbm.at[pl.ds(0, RPC)], idx_v[b],
                                  i_sems[b]).wait()
            pltpu.make_async_copy(xv_hbm.at[pl.ds(0, RPC)], xv_v[b],
                                  i_sems[b]).wait()

        def fire_rows(b):
            for j in range(NG):
                sl = pl.ds(j * GSLICE, GSLICE)
                pltpu.async_copy(emb2_hbm.at[idx_v[b].at[sl]],
                                 rows_v[b].at[sl], r_sems[b])
                pltpu.async_copy(emb1_hbm.at[idx_v[b].at[sl]],
                                 fo_v[b].at[sl], f_sems[b])

        def wait_rows(b):
            pltpu.make_async_copy(emb2_hbm.at[pl.ds(0, RPC)], rows_v[b],
                                  r_sems[b]).wait()
            pltpu.make_async_copy(emb1_hbm.at[pl.ds(0, RPC)], fo_v[b],
                                  f_sems[b]).wait()

        def fire_out(b, c):
            cbase = base + c * CHUNK
            pltpu.async_copy(sum_st[b], sum_hbm.at[pl.ds(cbase, CHUNK)],
                             o_sems[b])
            pltpu.async_copy(sq_st[b], sq_hbm.at[pl.ds(cbase, CHUNK)],
                             o_sems[b])
            pltpu.async_copy(fo_v[b], fo_hbm.at[pl.ds(cbase * F, RPC)],
                             o_sems[b])

        def wait_out(b):
            pltpu.make_async_copy(sum_st[b], sum_hbm.at[pl.ds(0, CHUNK)],
                                  o_sems[b]).wait()
            pltpu.make_async_copy(sq_st[b], sq_hbm.at[pl.ds(0, CHUNK)],
                                  o_sems[b]).wait()
            pltpu.make_async_copy(fo_v[b], fo_hbm.at[pl.ds(0, RPC)],
                                  o_sems[b]).wait()

        def compute(b):
            def sample_body(s, carry):
                accs = [jnp.zeros((L,), jnp.float32) for _ in range(E // L)]
                sqs = [jnp.zeros((L,), jnp.float32) for _ in range(E // L)]
                for f in range(F):
                    row = s * F + f
                    xv = plsc.load_gather(xv_v[b],
                                          [jnp.full((L,), row, jnp.int32)])
                    for e in range(E // L):
                        v = rows_v[b][row, pl.ds(e * L, L)] * xv
                        accs[e] = accs[e] + v
                        sqs[e] = sqs[e] + v * v
                for e in range(E // L):
                    sum_st[b][s, pl.ds(e * L, L)] = accs[e]
                    sq_st[b][s, pl.ds(e * L, L)] = sqs[e]
                return carry

            lax.fori_loop(0, CHUNK, sample_body, None)

        fire_idx(0, 0)
        fire_idx(1, 1)
        wait_idx(0)
        fire_rows(0)

        @pl.loop(0, n_chunk, step=2)
        def chunk_pair(c0):
            for bb in range(2):
                c = c0 + bb
                b = bb
                nb = 1 - bb
                @pl.when(c + 1 < n_chunk)
                def _():
                    wait_idx(nb)
                    pl.when(c >= 1)(lambda: wait_out(nb))
                    fire_rows(nb)

                wait_rows(b)
                compute(b)
                fire_out(b, c)

                @pl.when(c + 2 < n_chunk)
                def _():
                    fire_idx(b, c + 2)

        wait_out(0)
        wait_out(1)

    return sc_gather_reduce


BLK = 512


def _tc_body(sum_ref, sq_ref, fo_ref, xv_ref, w0_ref, b0_ref, w1_ref, b1_ref,
             w2_ref, b2_ref, bias_ref, out_ref):
    ones_e = jnp.ones((E, 1), jnp.float32)
    ones_h = jnp.ones((H, 1), jnp.float32)
    ones_f = jnp.ones((F, 1), jnp.float32)
    dn = (((1,), (0,)), ((), ()))
    dn_t = (((1,), (1,)), ((), ()))

    s = sum_ref[...]
    so = 0.5 * (s * s - sq_ref[...])
    fo = fo_ref[...] * xv_ref[...]
    fm = (lax.dot_general(fo, ones_f, dn, preferred_element_type=jnp.float32)
          + lax.dot_general(so, ones_e, dn, preferred_element_type=jnp.float32)
          + bias_ref[0])
    x = lax.dot_general(so, w0_ref[...], dn_t,
                        preferred_element_type=jnp.float32)
    x = jnp.maximum(x + b0_ref[...][None, :], 0.0)
    z1 = fm + lax.dot_general(x, ones_h, dn,
                              preferred_element_type=jnp.float32)
    x = lax.dot_general(x, w1_ref[...], dn_t,
                        preferred_element_type=jnp.float32)
    x = jnp.maximum(x + b1_ref[...][None, :], 0.0)
    z2 = fm + lax.dot_general(x, ones_h, dn,
                              preferred_element_type=jnp.float32)
    x = lax.dot_general(x, w2_ref[...], dn_t,
                        preferred_element_type=jnp.float32)
    x = jnp.maximum(x + b2_ref[...][None, :], 0.0)
    z3 = fm + lax.dot_general(x, ones_h, dn,
                              preferred_element_type=jnp.float32)
    out_ref[...] = jax.nn.sigmoid(jnp.concatenate([z1, z2, z3], axis=1))


def _make_tc_mlp(bs):
    return pl.pallas_call(
        _tc_body,
        grid=(bs // BLK,),
        in_specs=[
            pl.BlockSpec((BLK, E), lambda i: (i, 0)),
            pl.BlockSpec((BLK, E), lambda i: (i, 0)),
            pl.BlockSpec((BLK, F), lambda i: (i, 0)),
            pl.BlockSpec((BLK, F), lambda i: (i, 0)),
            pl.BlockSpec((H, E), lambda i: (0, 0)),
            pl.BlockSpec((H,), lambda i: (0,)),
            pl.BlockSpec((H, H), lambda i: (0, 0)),
            pl.BlockSpec((H,), lambda i: (0,)),
            pl.BlockSpec((H, H), lambda i: (0, 0)),
            pl.BlockSpec((H,), lambda i: (0,)),
            pl.BlockSpec(memory_space=pltpu.SMEM),
        ],
        out_specs=pl.BlockSpec((BLK, 3), lambda i: (i, 0)),
        out_shape=jax.ShapeDtypeStruct((bs, 3), jnp.float32),
    )


_sc_slice = _make_sc_gather_reduce(BS)
_tc_slice = _make_tc_mlp(BS)


def kernel(Xi, Xv, emb1, emb2, W0, b0, W1, b1, W2, b2, bias):
    idx = Xi[..., 0].astype(jnp.int32)
    gidx = (idx + (jnp.arange(F, dtype=jnp.int32) * V)[None, :]).reshape(B * F)
    xv_flat = Xv.reshape(B * F)
    emb1_flat = emb1.reshape(F * V)
    emb2_flat = emb2.reshape(F * V, E)
    def sc_call(k):
        lo = k * BS * F
        return _sc_slice(
            lax.dynamic_slice(gidx, (lo,), (BS * F,)),
            lax.dynamic_slice(xv_flat, (lo,), (BS * F,)),
            emb1_flat, emb2_flat)

    def tc_call(k, sc_out):
        sum_k, sq_k, fo_k = sc_out
        return _tc_slice(sum_k, sq_k, fo_k.reshape(BS, F),
                         lax.dynamic_slice(Xv, (k * BS, 0), (BS, F)),
                         W0, b0, W1, b1, W2, b2, bias)

    sc_prev = sc_call(0)
    outs = []
    for k in range(NSLICE):
        sc_next = sc_call(k + 1) if k + 1 < NSLICE else None
        outs.append(tc_call(k, sc_prev))
        sc_prev = sc_next
    preds = jnp.concatenate(outs, axis=0).T
    return preds[2], preds


# speedup vs baseline: 1.0493x; 1.0493x over previous
import functools

import jax
import jax.numpy as jnp
from jax import lax
from jax.experimental import pallas as pl
from jax.experimental.pallas import tpu as pltpu
from jax.experimental.pallas import tpu_sc as plsc

B, F, V, E, H = 16384, 26, 100000, 128, 512
NC, NS, L = 2, 16, 16
NW = NC * NS
NSLICE = 4
BS = B // NSLICE
CHUNK = 16
RPC = CHUNK * F
GSLICE = 104
NG = RPC // GSLICE
CPAD = RPC + L

_sc_mesh = plsc.VectorSubcoreMesh(core_axis_name="c", subcore_axis_name="s")


def _make_sc_gather_reduce(bs, off):
    s_per_w = bs // NW
    n_chunk = s_per_w // CHUNK

    @functools.partial(
        pl.kernel,
        out_type=(
            jax.ShapeDtypeStruct((bs, E), jnp.float32),
            jax.ShapeDtypeStruct((bs, E), jnp.float32),
            jax.ShapeDtypeStruct((bs, E), jnp.float32),
        ),
        mesh=_sc_mesh,
        compiler_params=pltpu.CompilerParams(needs_layout_passes=False),
        scratch_types=[
            pltpu.VMEM((CHUNK * E,), jnp.int32),
            pltpu.VMEM((CHUNK * E,), jnp.int32),
            pltpu.VMEM((CPAD,), jnp.int32),
            pltpu.VMEM((CPAD,), jnp.int32),
            pltpu.VMEM((CHUNK * E,), jnp.float32),
            pltpu.VMEM((CHUNK * E,), jnp.float32),
            pltpu.VMEM((CPAD,), jnp.float32),
            pltpu.VMEM((CPAD,), jnp.float32),
            pltpu.VMEM((RPC, E), jnp.float32),
            pltpu.VMEM((RPC, E), jnp.float32),
            pltpu.VMEM((CHUNK, E), jnp.float32),
            pltpu.VMEM((CHUNK, E), jnp.float32),
            pltpu.VMEM((CHUNK, E), jnp.float32),
            pltpu.VMEM((CHUNK, E), jnp.float32),
            pltpu.VMEM((CHUNK, E), jnp.float32),
            pltpu.VMEM((CHUNK, E), jnp.float32),
            pltpu.SemaphoreType.DMA,
            pltpu.SemaphoreType.DMA,
            pltpu.SemaphoreType.DMA,
            pltpu.SemaphoreType.DMA,
            pltpu.SemaphoreType.DMA,
            pltpu.SemaphoreType.DMA,
            pltpu.SemaphoreType.DMA,
            pltpu.SemaphoreType.DMA,
        ],
    )
    def sc_gather_reduce(gidx_hbm, xv_hbm, emb1_hbm, emb2_hbm,
                         sum_hbm, sq_hbm, fo_hbm,
                         idx_v0, idx_v1, cidx_v0, cidx_v1, xv_v0, xv_v1,
                         fo_v0, fo_v1, rows_v0, rows_v1,
                         sum_st0, sum_st1, sq_st0, sq_st1, fo_st0, fo_st1,
                         i_sem0, i_sem1, r_sem0, r_sem1, f_sem0, f_sem1,
                         o_sem0, o_sem1):
        idx_v = (idx_v0, idx_v1)
        cidx_v = (cidx_v0, cidx_v1)
        xv_v = (xv_v0, xv_v1)
        fo_v = (fo_v0, fo_v1)
        rows_v = (rows_v0, rows_v1)
        sum_st = (sum_st0, sum_st1)
        sq_st = (sq_st0, sq_st1)
        fo_st = (fo_st0, fo_st1)
        i_sems = (i_sem0, i_sem1)
        r_sems = (r_sem0, r_sem1)
        f_sems = (f_sem0, f_sem1)
        o_sems = (o_sem0, o_sem1)
        wid = lax.axis_index("s") * NC + lax.axis_index("c")
        base = off + wid * s_per_w
        zeros16 = jnp.zeros((L,), jnp.float32)
        iota16 = jnp.arange(L, dtype=jnp.int32)

        for b in range(2):
            fo_v[b][pl.ds(RPC, L)] = zeros16
            for s in range(CHUNK):
                for j in range(2, E // L):
                    fo_st[b][s, pl.ds(j * L, L)] = zeros16

        def fire_idx(b, c):
            e0 = (base + c * CHUNK) * E
            pltpu.async_copy(gidx_hbm.at[pl.ds(e0, CHUNK * E)], idx_v[b],
                             i_sems[b])
            pltpu.async_copy(xv_hbm.at[pl.ds(e0, CHUNK * E)], xv_v[b],
                             i_sems[b])

        def wait_idx(b):
            pltpu.make_async_copy(gidx_hbm.at[pl.ds(0, CHUNK * E)], idx_v[b],
                                  i_sems[b]).wait()
            pltpu.make_async_copy(xv_hbm.at[pl.ds(0, CHUNK * E)], xv_v[b],
                                  i_sems[b]).wait()

        def compact(b):
            for s in range(CHUNK):
                va = idx_v[b][pl.ds(s * E, L)]
                vb = idx_v[b][pl.ds(s * E + L, L)]
                plsc.store_scatter(cidx_v[b], [iota16 + (s * F)], va)
                plsc.store_scatter(cidx_v[b], [iota16 + (s * F + L)], vb)

        def fire_rows(b):
            for j in range(NG):
                sl = pl.ds(j * GSLICE, GSLICE)
                pltpu.async_copy(emb2_hbm.at[cidx_v[b].at[sl]],
                                 rows_v[b].at[sl], r_sems[b])
                pltpu.async_copy(emb1_hbm.at[cidx_v[b].at[sl]],
                                 fo_v[b].at[sl], f_sems[b])

        def wait_rows(b):
            pltpu.make_async_copy(emb2_hbm.at[pl.ds(0, RPC)], rows_v[b],
                                  r_sems[b]).wait()
            pltpu.make_async_copy(emb1_hbm.at[pl.ds(0, RPC)],
                                  fo_v[b].at[pl.ds(0, RPC)],
                                  f_sems[b]).wait()

        def fire_out(b, c):
            r0 = base + c * CHUNK
            pltpu.async_copy(sum_st[b], sum_hbm.at[pl.ds(r0, CHUNK)],
                             o_sems[b])
            pltpu.async_copy(sq_st[b], sq_hbm.at[pl.ds(r0, CHUNK)], o_sems[b])
            pltpu.async_copy(fo_st[b], fo_hbm.at[pl.ds(r0, CHUNK)], o_sems[b])

        def wait_out(b):
            pltpu.make_async_copy(sum_st[b], sum_hbm.at[pl.ds(0, CHUNK)],
                                  o_sems[b]).wait()
            pltpu.make_async_copy(sq_st[b], sq_hbm.at[pl.ds(0, CHUNK)],
                                  o_sems[b]).wait()
            pltpu.make_async_copy(fo_st[b], fo_hbm.at[pl.ds(0, CHUNK)],
                                  o_sems[b]).wait()

        def compute(b):
            def sample_body(s, carry):
                accs = [jnp.zeros((L,), jnp.float32) for _ in range(E // L)]
                sqs = [jnp.zeros((L,), jnp.float32) for _ in range(E // L)]
                sbase = s * E
                for f in range(F):
                    row = s * F + f
                    xv = plsc.load_gather(
                        xv_v[b], [jnp.full((L,), sbase + f, jnp.int32)])
                    for e in range(E // L):
                        v = rows_v[b][row, pl.ds(e * L, L)] * xv
                        accs[e] = accs[e] + v
                        sqs[e] = sqs[e] + v * v
                for e in range(E // L):
                    sum_st[b][s, pl.ds(e * L, L)] = accs[e]
                    sq_st[b][s, pl.ds(e * L, L)] = sqs[e]
                cb = s * F
                fo_st[b][s, pl.ds(0, L)] = (
                    plsc.load_gather(fo_v[b], [iota16 + cb])
                    * plsc.load_gather(xv_v[b], [iota16 + sbase]))
                fo_st[b][s, pl.ds(L, L)] = (
                    plsc.load_gather(fo_v[b], [iota16 + (cb + L)])
                    * plsc.load_gather(xv_v[b], [iota16 + (sbase + L)]))
                return carry

            lax.fori_loop(0, CHUNK, sample_body, None)

        fire_idx(0, 0)
        fire_idx(1, 1)
        wait_idx(0)
        compact(0)
        fire_rows(0)

        @pl.loop(0, n_chunk, step=2)
        def chunk_pair(c0):
            for bb in range(2):
                c = c0 + bb
                b = bb
                nb = 1 - bb
                @pl.when(c + 1 < n_chunk)
                def _():
                    wait_idx(nb)
                    compact(nb)
                    pl.when(c >= 1)(lambda: wait_out(nb))
                    fire_rows(nb)

                wait_rows(b)
                compute(b)
                fire_out(b, c)

                @pl.when(c + 2 < n_chunk)
                def _():
                    fire_idx(b, c + 2)

        wait_out(0)
        wait_out(1)

    return sc_gather_reduce


BLK = 512


def _tc_body(sum_ref, sq_ref, fo_ref, w0_ref, b0_ref, w1_ref, b1_ref,
             w2_ref, b2_ref, bias_ref, out_ref):
    ones_e = jnp.ones((E, 1), jnp.float32)
    ones_h = jnp.ones((H, 1), jnp.float32)
    dn = (((1,), (0,)), ((), ()))
    dn_t = (((1,), (1,)), ((), ()))

    s = sum_ref[...]
    so = 0.5 * (s * s - sq_ref[...])
    fm = (lax.dot_general(so + fo_ref[...], ones_e, dn,
                          preferred_element_type=jnp.float32)
          + bias_ref[0])
    x = lax.dot_general(so, w0_ref[...], dn_t,
                        preferred_element_type=jnp.float32)
    x = jnp.maximum(x + b0_ref[...][None, :], 0.0)
    z1 = fm + lax.dot_general(x, ones_h, dn,
                              preferred_element_type=jnp.float32)
    x = lax.dot_general(x, w1_ref[...], dn_t,
                        preferred_element_type=jnp.float32)
    x = jnp.maximum(x + b1_ref[...][None, :], 0.0)
    z2 = fm + lax.dot_general(x, ones_h, dn,
                              preferred_element_type=jnp.float32)
    x = lax.dot_general(x, w2_ref[...], dn_t,
                        preferred_element_type=jnp.float32)
    x = jnp.maximum(x + b2_ref[...][None, :], 0.0)
    z3 = fm + lax.dot_general(x, ones_h, dn,
                              preferred_element_type=jnp.float32)
    out_ref[...] = jax.nn.sigmoid(jnp.concatenate([z1, z2, z3], axis=1))


def _make_tc_mlp(bs):
    return pl.pallas_call(
        _tc_body,
        grid=(bs // BLK,),
        in_specs=[
            pl.BlockSpec((BLK, E), lambda i: (i, 0)),
            pl.BlockSpec((BLK, E), lambda i: (i, 0)),
            pl.BlockSpec((BLK, E), lambda i: (i, 0)),
            pl.BlockSpec((H, E), lambda i: (0, 0)),
            pl.BlockSpec((H,), lambda i: (0,)),
            pl.BlockSpec((H, H), lambda i: (0, 0)),
            pl.BlockSpec((H,), lambda i: (0,)),
            pl.BlockSpec((H, H), lambda i: (0, 0)),
            pl.BlockSpec((H,), lambda i: (0,)),
            pl.BlockSpec(memory_space=pltpu.SMEM),
        ],
        out_specs=pl.BlockSpec((BLK, 3), lambda i: (i, 0)),
        out_shape=jax.ShapeDtypeStruct((bs, 3), jnp.float32),
    )


_sc_slices = [_make_sc_gather_reduce(BS, k * BS) for k in range(NSLICE)]
_tc_slice = _make_tc_mlp(BS)


def kernel(Xi, Xv, emb1, emb2, W0, b0, W1, b1, W2, b2, bias):
    idx = Xi[..., 0].astype(jnp.int32)
    gidx = idx + (jnp.arange(F, dtype=jnp.int32) * V)[None, :]
    pad = ((0, 0), (0, E - F))
    gidx_pad = jnp.pad(gidx, pad).reshape(B * E)
    xv_pad = jnp.pad(Xv, pad).reshape(B * E)
    emb1_flat = emb1.reshape(F * V)
    emb2_flat = emb2.reshape(F * V, E)

    def tc_call(k, sc_out):
        sum_k, sq_k, fo_k = sc_out
        return _tc_slice(sum_k, sq_k, fo_k,
                         W0, b0, W1, b1, W2, b2, bias)

    sc_prev = _sc_slices[0](gidx_pad, xv_pad, emb1_flat, emb2_flat)
    outs = []
    for k in range(NSLICE):
        sc_next = (_sc_slices[k + 1](gidx_pad, xv_pad, emb1_flat, emb2_flat)
                   if k + 1 < NSLICE else None)
        outs.append(tc_call(k, sc_prev))
        sc_prev = sc_next
    preds = jnp.concatenate(outs, axis=0).T
    return preds[2], preds
